# baseline (speedup 1.0000x reference)
import jax
import jax.numpy as jnp
from jax import lax
from jax.experimental import pallas as pl
from jax.experimental.pallas import tpu as pltpu

N_TOK = 32768
D_MODEL = 768
N_EXPERTS = 64
TOP_K = 8
BT = 4096


def _body(x_ref, w_ref, b_ref, scores_ref, g_ref, i_ref):
    x = x_ref[...]
    w = w_ref[...]
    logits = jnp.dot(x, w, preferred_element_type=jnp.float32) + b_ref[...]
    scores = jax.nn.sigmoid(logits)
    scores_ref[...] = scores

    work = scores.T

    e_iota = lax.broadcasted_iota(jnp.int32, (N_EXPERTS, BT), 0).astype(jnp.float32)
    rev_key = (N_EXPERTS - 1.0) - e_iota
    vals = []
    idxs = []
    for _ in range(TOP_K):
        m = jnp.max(work, axis=0, keepdims=True)
        is_max = work >= m
        nk = jnp.max(jnp.where(is_max, rev_key, -1.0), axis=0, keepdims=True)
        idx = (N_EXPERTS - 1.0) - nk
        vals.append(m)
        idxs.append(idx)
        work = jnp.where(e_iota == idx, -1.0, work)

    total = vals[0]
    for k in range(1, TOP_K):
        total = total + vals[k]
    inv = 1.0 / total
    g_t = jnp.concatenate([v * inv for v in vals], axis=0)
    i_t = jnp.concatenate(idxs, axis=0)
    g_ref[...] = g_t.T
    i_ref[...] = i_t.T.astype(jnp.int32)


def kernel(x, W, b):
    grid = (N_TOK // BT,)
    scores, g, i = pl.pallas_call(
        _body,
        grid=grid,
        in_specs=[
            pl.BlockSpec((BT, D_MODEL), lambda t: (t, 0)),
            pl.BlockSpec((D_MODEL, N_EXPERTS), lambda t: (0, 0)),
            pl.BlockSpec((1, N_EXPERTS), lambda t: (0, 0)),
        ],
        out_specs=[
            pl.BlockSpec((BT, N_EXPERTS), lambda t: (t, 0)),
            pl.BlockSpec((BT, TOP_K), lambda t: (t, 0)),
            pl.BlockSpec((BT, TOP_K), lambda t: (t, 0)),
        ],
        out_shape=[
            jax.ShapeDtypeStruct((N_TOK, N_EXPERTS), jnp.float32),
            jax.ShapeDtypeStruct((N_TOK, TOP_K), jnp.float32),
            jax.ShapeDtypeStruct((N_TOK, TOP_K), jnp.int32),
        ],
    )(x, W, b.reshape(1, N_EXPERTS))
    return (g, i, scores)


# speedup vs baseline: 1.4794x; 1.4794x over previous
import jax
import jax.numpy as jnp
from jax import lax
from jax.experimental import pallas as pl
from jax.experimental.pallas import tpu as pltpu

N_TOK = 32768
D_MODEL = 768
N_EXPERTS = 64
TOP_K = 8
BT = 4096


def _body(x_ref, w_ref, b_ref, scores_ref, g_ref, i_ref):
    x = x_ref[...]
    w = w_ref[...]
    logits = jnp.dot(x, w, preferred_element_type=jnp.float32) + b_ref[...]
    scores = jax.nn.sigmoid(logits)
    scores_ref[...] = scores

    work = scores.T

    e_iota = lax.broadcasted_iota(jnp.int32, (N_EXPERTS, BT), 0).astype(jnp.float32)
    rev_key = (N_EXPERTS - 1.0) - e_iota
    vals = []
    idxs = []
    for _ in range(TOP_K):
        m = jnp.max(work, axis=0, keepdims=True)
        is_max = work >= m
        nk = jnp.max(jnp.where(is_max, rev_key, -1.0), axis=0, keepdims=True)
        idx = (N_EXPERTS - 1.0) - nk
        vals.append(m)
        idxs.append(idx)
        work = jnp.where(e_iota == idx, -1.0, work)

    total = vals[0]
    for k in range(1, TOP_K):
        total = total + vals[k]
    inv = 1.0 / total
    g_t = jnp.concatenate([v * inv for v in vals], axis=0)
    i_t = jnp.concatenate(idxs, axis=0).astype(jnp.int32)
    g_ref[...] = g_t
    i_ref[...] = i_t


def kernel(x, W, b):
    grid = (N_TOK // BT,)
    scores, g_t, i_t = pl.pallas_call(
        _body,
        grid=grid,
        in_specs=[
            pl.BlockSpec((BT, D_MODEL), lambda t: (t, 0)),
            pl.BlockSpec((D_MODEL, N_EXPERTS), lambda t: (0, 0)),
            pl.BlockSpec((1, N_EXPERTS), lambda t: (0, 0)),
        ],
        out_specs=[
            pl.BlockSpec((BT, N_EXPERTS), lambda t: (t, 0)),
            pl.BlockSpec((TOP_K, BT), lambda t: (0, t)),
            pl.BlockSpec((TOP_K, BT), lambda t: (0, t)),
        ],
        out_shape=[
            jax.ShapeDtypeStruct((N_TOK, N_EXPERTS), jnp.float32),
            jax.ShapeDtypeStruct((TOP_K, N_TOK), jnp.float32),
            jax.ShapeDtypeStruct((TOP_K, N_TOK), jnp.int32),
        ],
        compiler_params=pltpu.CompilerParams(dimension_semantics=("arbitrary",)),
    )(x, W, b.reshape(1, N_EXPERTS))
    return (g_t.T, i_t.T, scores)
